# Initial kernel scaffold; baseline (speedup 1.0000x reference)
import jax
import jax.numpy as jnp
from jax.experimental import pallas as pl


def kernel(x, pos, y, theta, W1, b1, W2, b2, edge_index, edge_attr):
    raise NotImplementedError("write your pallas kernel here")


# speedup vs baseline: 41.5247x; 41.5247x over previous
import functools

import jax
import jax.numpy as jnp
from jax import lax
from jax.experimental import pallas as pl
from jax.experimental.pallas import tpu as pltpu
from jax.experimental.pallas import tpu_sc as plsc

N = 50000
K = 20
NP8 = 6272
NROW = N // 8
NPAD = 8 * NP8
KNP = K * NPAD
CHUNK = 512
EPAD = 1984 * CHUNK
NCHUNK_PER_SUB = 124
EPS = 1e-12


def _sqrt16(t):
  bits = plsc.bitcast(t, jnp.int32)
  h = jnp.int32(0x5F3759DF) - lax.shift_right_logical(bits, 1)
  r = plsc.bitcast(h, jnp.float32)
  half_t = 0.5 * t
  for _ in range(3):
    r = r * (1.5 - half_t * r * r)
  return t * r


def _mlp_body(x_ref, w1_ref, b1_ref, w2t_ref, b2_ref, o_ref):
  h = jnp.dot(x_ref[...], w1_ref[...], preferred_element_type=jnp.float32)
  h = jnp.maximum(h + b1_ref[...], 0.0)
  o = lax.dot_general(w2t_ref[...], h, (((1,), (1,)), ((), ())),
                      preferred_element_type=jnp.float32)
  o_ref[...] = o + b2_ref[0, 0]


def _mlp(x, w1, b1, w2t, b2):
  bn = 512
  grid = (pl.cdiv(N, bn),)
  return pl.pallas_call(
      _mlp_body,
      grid=grid,
      in_specs=[
          pl.BlockSpec((bn, 128), lambda i: (i, 0)),
          pl.BlockSpec((128, 128), lambda i: (0, 0)),
          pl.BlockSpec((1, 128), lambda i: (0, 0)),
          pl.BlockSpec((1, 128), lambda i: (0, 0)),
          pl.BlockSpec(memory_space=pltpu.SMEM),
      ],
      out_specs=pl.BlockSpec((1, bn), lambda i: (0, i)),
      out_shape=jax.ShapeDtypeStruct((1, N), jnp.float32),
  )(x, w1, b1, w2t, b2)


KC4 = K * 4 * NP8


KC2 = K * 2 * NP8
KC1 = K * NP8


def _sc1_body(src_h, dst_h, slot_h, posx_h, posy_h, sp_h,
              cov_o, px_o, py_o,
              posx_v, posy_v, sbuf, dbuf, lbuf,
              covb, pxb, pyb, idxb, sp_v, zbuf, bnc,
              acc_cov, acc_px, acc_py):
  core = lax.axis_index("c")
  sub = lax.axis_index("s")

  pltpu.sync_copy(posx_h, posx_v)
  pltpu.sync_copy(posy_h, posy_v)
  pltpu.sync_copy(sp_h, sp_v)

  sig = sp_v[pl.ds(0, 16)]
  phi = sp_v[pl.ds(16, 16)]

  def _zb(i):
    zbuf[pl.ds(i * 16, 16)] = jnp.zeros((16,), jnp.float32)
  lax.fori_loop(0, NP8 // 16, lambda i, _: (_zb(i), 0)[1], 0)

  def _pass(p, _):
    for kk in range(10):
      @pl.when(sub == kk % 16)
      def _(kk=kk):
        pltpu.sync_copy(zbuf, acc_cov.at[pl.ds(kk * NP8, NP8)])
        pltpu.sync_copy(zbuf, acc_px.at[pl.ds(kk * NP8, NP8)])
        pltpu.sync_copy(zbuf, acc_py.at[pl.ds(kk * NP8, NP8)])
    plsc.subcore_barrier()

    row = p // 2
    half = p - 2 * row
    lvbase = 10 * half
    rbase = core * 4 + row

    def _chunk(g, _):
      off = (g * 16 + sub) * CHUNK
      pltpu.sync_copy(src_h.at[pl.ds(off, CHUNK)], sbuf)
      pltpu.sync_copy(dst_h.at[pl.ds(off, CHUNK)], dbuf)
      pltpu.sync_copy(slot_h.at[pl.ds(off, CHUNK)], lbuf)

      def _vec(j, _):
        sl = pl.ds(j * 16, 16)
        sv = sbuf[sl]
        dv = dbuf[sl]
        lv = lbuf[sl]
        dg = jnp.minimum(dv, N - 1)
        xs = plsc.load_gather(posx_v, [sv])
        ys = plsc.load_gather(posy_v, [sv])
        xd = plsc.load_gather(posx_v, [dg])
        yd = plsc.load_gather(posy_v, [dg])
        dx = xd - xs
        dy = yd - ys
        t = dx * dx + dy * dy + EPS
        dist = _sqrt16(t)
        cov = sig * jnp.exp(-phi * dist)
        r8 = dv // NROW
        c6 = dv - r8 * NROW
        lvr = lv - lvbase
        ok = (r8 == rbase) & (lvr >= 0) & (lvr < 10)
        idx = lvr * NP8 + c6
        idx = jnp.where(ok, idx, -1)
        covb[sl] = cov
        pxb[sl] = xs
        pyb[sl] = ys
        idxb[sl] = idx
        return 0

      lax.fori_loop(0, CHUNK // 16, _vec, 0)
      ind = plsc.Indices(idxb, ignored_value=-1)
      pltpu.sync_copy(covb, acc_cov.at[ind], add=True)
      pltpu.sync_copy(pxb, acc_px.at[ind], add=True)
      pltpu.sync_copy(pyb, acc_py.at[ind], add=True)
      return 0

    lax.fori_loop(0, NCHUNK_PER_SUB, _chunk, 0)
    plsc.subcore_barrier()

    for kk in range(10):
      @pl.when(sub == kk % 16)
      def _(kk=kk):
        dsto = (kk + lvbase) * 8 * NP8 + rbase * NP8
        pltpu.sync_copy(acc_cov.at[pl.ds(kk * NP8, NP8)], bnc)
        pltpu.sync_copy(bnc, cov_o.at[pl.ds(dsto, NP8)])
        pltpu.sync_copy(acc_px.at[pl.ds(kk * NP8, NP8)], bnc)
        pltpu.sync_copy(bnc, px_o.at[pl.ds(dsto, NP8)])
        pltpu.sync_copy(acc_py.at[pl.ds(kk * NP8, NP8)], bnc)
        pltpu.sync_copy(bnc, py_o.at[pl.ds(dsto, NP8)])
    plsc.subcore_barrier()
    return 0

  lax.fori_loop(0, 8, _pass, 0)


def _sc1(src, dst, slot, posx, posy, scal):
  mesh = plsc.VectorSubcoreMesh(core_axis_name="c", subcore_axis_name="s")
  f32 = jnp.float32
  kern = pl.kernel(
      _sc1_body,
      out_type=[jax.ShapeDtypeStruct((KNP,), f32)] * 3,
      mesh=mesh,
      compiler_params=pltpu.CompilerParams(needs_layout_passes=False),
      scratch_types=[
          pltpu.VMEM((N,), f32), pltpu.VMEM((N,), f32),
          pltpu.VMEM((CHUNK,), jnp.int32), pltpu.VMEM((CHUNK,), jnp.int32),
          pltpu.VMEM((CHUNK,), jnp.int32),
          pltpu.VMEM((CHUNK,), f32), pltpu.VMEM((CHUNK,), f32),
          pltpu.VMEM((CHUNK,), f32), pltpu.VMEM((CHUNK,), jnp.int32),
          pltpu.VMEM((32,), f32),
          pltpu.VMEM((NP8,), f32), pltpu.VMEM((NP8,), f32),
          pltpu.VMEM_SHARED((10 * NP8,), f32),
          pltpu.VMEM_SHARED((10 * NP8,), f32),
          pltpu.VMEM_SHARED((10 * NP8,), f32),
      ],
  )
  return kern(src, dst, slot, posx, posy, scal)


def _minmax_body(px_ref, py_ref, mx_ref, mn_ref):
  i = pl.program_id(0)
  lanes = i * 128 + lax.broadcasted_iota(jnp.int32, (K, 8, 128), 2)
  valid = lanes < NROW
  big = jnp.float32(3.4e38)
  px = px_ref[...]
  py = py_ref[...]
  bmax = jnp.maximum(jnp.max(jnp.where(valid, px, -big)),
                     jnp.max(jnp.where(valid, py, -big)))
  bmin = jnp.minimum(jnp.min(jnp.where(valid, px, big)),
                     jnp.min(jnp.where(valid, py, big)))

  @pl.when(i == 0)
  def _():
    mx_ref[0, 0] = -big
    mn_ref[0, 0] = big

  mx_ref[0, 0] = jnp.maximum(mx_ref[0, 0], bmax)
  mn_ref[0, 0] = jnp.minimum(mn_ref[0, 0], bmin)


def _minmax(px3, py3):
  grid = (NP8 // 128,)
  return pl.pallas_call(
      _minmax_body,
      grid=grid,
      in_specs=[
          pl.BlockSpec((K, 8, 128), lambda i: (0, 0, i)),
          pl.BlockSpec((K, 8, 128), lambda i: (0, 0, i)),
      ],
      out_specs=[
          pl.BlockSpec(memory_space=pltpu.SMEM),
          pl.BlockSpec(memory_space=pltpu.SMEM),
      ],
      out_shape=[jax.ShapeDtypeStruct((1, 1), jnp.float32)] * 2,
  )(px3, py3)


def _chol_body(th_ref, mx_ref, mn_ref, cov_ref, px_ref, py_ref, rf_ref,
               b_ref, f_ref):
  sig = th_ref[0, 0]
  phi = th_ref[0, 1]
  tsq = th_ref[0, 2]
  t02 = th_ref[0, 3]
  scale = 10000.0 * (mx_ref[0, 0] - mn_ref[0, 0])

  fx = []
  fy = []
  for a in range(K):
    cx = px_ref[a]
    cy = py_ref[a]
    fx.append(jnp.where(cx == 0.0, rf_ref[2 * a] * scale, cx))
    fy.append(jnp.where(cy == 0.0, rf_ref[2 * a + 1] * scale, cy))

  m = [[None] * K for _ in range(K)]
  for a in range(K):
    for b in range(a + 1):
      dx = fx[a] - fx[b]
      dy = fy[a] - fy[b]
      dist = jnp.sqrt(dx * dx + dy * dy + EPS)
      v = sig * jnp.exp(-phi * dist)
      if a == b:
        v = v + tsq
      m[a][b] = v

  inv = [None] * K
  for j in range(K):
    inv[j] = 1.0 / jnp.sqrt(m[j][j])
    for r in range(j + 1, K):
      m[r][j] = m[r][j] * inv[j]
    for r in range(j + 1, K):
      for b in range(j + 1, r + 1):
        m[r][b] = m[r][b] - m[r][j] * m[b][j]

  z = [None] * K
  for j in range(K):
    acc = cov_ref[j]
    for p in range(j):
      acc = acc - m[j][p] * z[p]
    z[j] = acc * inv[j]

  f = t02
  for j in range(K):
    f = f - z[j] * z[j]
  f_ref[...] = jnp.maximum(f, 1e-6)

  bb = [None] * K
  for j in range(K - 1, -1, -1):
    acc = z[j]
    for p in range(j + 1, K):
      acc = acc - m[p][j] * bb[p]
    bb[j] = acc * inv[j]
  for j in range(K):
    b_ref[j] = bb[j]


def _chol(th, mx, mn, cov3, px3, py3, rf3):
  grid = (NP8 // 128,)
  return pl.pallas_call(
      _chol_body,
      grid=grid,
      in_specs=[
          pl.BlockSpec(memory_space=pltpu.SMEM),
          pl.BlockSpec(memory_space=pltpu.SMEM),
          pl.BlockSpec(memory_space=pltpu.SMEM),
          pl.BlockSpec((K, 8, 128), lambda i: (0, 0, i)),
          pl.BlockSpec((K, 8, 128), lambda i: (0, 0, i)),
          pl.BlockSpec((K, 8, 128), lambda i: (0, 0, i)),
          pl.BlockSpec((2 * K, 8, 128), lambda i: (0, 0, i)),
      ],
      out_specs=[
          pl.BlockSpec((K, 8, 128), lambda i: (0, 0, i)),
          pl.BlockSpec((8, 128), lambda i: (0, i)),
      ],
      out_shape=[
          jax.ShapeDtypeStruct((K, 8, NP8), jnp.float32),
          jax.ShapeDtypeStruct((8, NP8), jnp.float32),
      ],
  )(th, mx, mn, cov3, px3, py3, rf3)


def _sc2_body(src_h, dst_h, slot_h, y_h, o_h, b_h,
              accy_o, acco_o,
              y_v, o_v, sbuf, dbuf, lbuf,
              gidx, sidx, gbuf, cyb, cob, zbuf, bnc,
              b_s, accy_s, acco_s):
  core = lax.axis_index("c")
  sub = lax.axis_index("s")

  pltpu.sync_copy(y_h, y_v)
  pltpu.sync_copy(o_h, o_v)

  def _zb(i):
    zbuf[pl.ds(i * 16, 16)] = jnp.zeros((16,), jnp.float32)
  lax.fori_loop(0, NP8 // 16, lambda i, _: (_zb(i), 0)[1], 0)

  for i in range(4):
    @pl.when(sub == i)
    def _(i=i):
      pltpu.sync_copy(zbuf, accy_s.at[pl.ds(i * NP8, NP8)])

    @pl.when(sub == 8 + i)
    def _(i=i):
      pltpu.sync_copy(zbuf, acco_s.at[pl.ds(i * NP8, NP8)])
  plsc.subcore_barrier()

  cbase = core * 4

  def _qpass(q, _):
    plsc.subcore_barrier()
    for kk in range(K):
      @pl.when(sub == kk % 16)
      def _(kk=kk):
        pltpu.sync_copy(
            b_h.at[pl.ds(kk * 8 * NP8 + (cbase + q) * NP8, NP8)], bnc)
        pltpu.sync_copy(bnc, b_s.at[pl.ds(kk * NP8, NP8)])
    plsc.subcore_barrier()

    def _chunk(g, _):
      off = (g * 16 + sub) * CHUNK
      pltpu.sync_copy(src_h.at[pl.ds(off, CHUNK)], sbuf)
      pltpu.sync_copy(dst_h.at[pl.ds(off, CHUNK)], dbuf)
      pltpu.sync_copy(slot_h.at[pl.ds(off, CHUNK)], lbuf)

      def _vec(j, _):
        sl = pl.ds(j * 16, 16)
        sv = sbuf[sl]
        dv = dbuf[sl]
        lv = lbuf[sl]
        r8 = dv // NROW
        c6 = dv - r8 * NROW
        ok = r8 == cbase + q
        gi = jnp.where(ok, lv * NP8 + c6, 0)
        si = jnp.where(ok, q * NP8 + c6, -1)
        gidx[sl] = gi
        sidx[sl] = si
        cyb[sl] = plsc.load_gather(y_v, [sv])
        cob[sl] = plsc.load_gather(o_v, [sv])
        return 0

      lax.fori_loop(0, CHUNK // 16, _vec, 0)
      pltpu.sync_copy(b_s.at[plsc.Indices(gidx)], gbuf)

      def _mul(j, _):
        sl = pl.ds(j * 16, 16)
        bv = gbuf[sl]
        cyb[sl] = bv * cyb[sl]
        cob[sl] = bv * cob[sl]
        return 0

      lax.fori_loop(0, CHUNK // 16, _mul, 0)
      ind = plsc.Indices(sidx, ignored_value=-1)
      pltpu.sync_copy(cyb, accy_s.at[ind], add=True)
      pltpu.sync_copy(cob, acco_s.at[ind], add=True)
      return 0

    lax.fori_loop(0, NCHUNK_PER_SUB, _chunk, 0)
    return 0

  lax.fori_loop(0, 4, _qpass, 0)
  plsc.subcore_barrier()

  for i in range(4):
    @pl.when(sub == i)
    def _(i=i):
      pltpu.sync_copy(accy_s.at[pl.ds(i * NP8, NP8)], zbuf)
      pltpu.sync_copy(zbuf, accy_o.at[pl.ds(core * 4 * NP8 + i * NP8, NP8)])

    @pl.when(sub == 8 + i)
    def _(i=i):
      pltpu.sync_copy(acco_s.at[pl.ds(i * NP8, NP8)], zbuf)
      pltpu.sync_copy(zbuf, acco_o.at[pl.ds(core * 4 * NP8 + i * NP8, NP8)])


def _sc2(src, dst, slot, y, o, bflat):
  mesh = plsc.VectorSubcoreMesh(core_axis_name="c", subcore_axis_name="s")
  f32 = jnp.float32
  kern = pl.kernel(
      _sc2_body,
      out_type=[jax.ShapeDtypeStruct((NPAD,), f32)] * 2,
      mesh=mesh,
      compiler_params=pltpu.CompilerParams(needs_layout_passes=False),
      scratch_types=[
          pltpu.VMEM((N,), f32), pltpu.VMEM((N,), f32),
          pltpu.VMEM((CHUNK,), jnp.int32), pltpu.VMEM((CHUNK,), jnp.int32),
          pltpu.VMEM((CHUNK,), jnp.int32),
          pltpu.VMEM((CHUNK,), jnp.int32), pltpu.VMEM((CHUNK,), jnp.int32),
          pltpu.VMEM((CHUNK,), f32), pltpu.VMEM((CHUNK,), f32),
          pltpu.VMEM((CHUNK,), f32),
          pltpu.VMEM((NP8,), f32), pltpu.VMEM((NP8,), f32),
          pltpu.VMEM_SHARED((K * NP8,), f32),
          pltpu.VMEM_SHARED((4 * NP8,), f32),
          pltpu.VMEM_SHARED((4 * NP8,), f32),
      ],
  )
  return kern(src, dst, slot, y, o, bflat)


def _decor_body(y_ref, o_ref, ay_ref, ao_ref, f_ref, yd_ref, od_ref):
  rs = jnp.sqrt(f_ref[...])
  yd_ref[...] = (y_ref[...] - ay_ref[...]) / rs
  od_ref[...] = (o_ref[...] - ao_ref[...]) / rs


def _decor(y2, o2, ay2, ao2, f2):
  grid = (NP8 // 128,)
  spec = pl.BlockSpec((8, 128), lambda i: (0, i))
  return pl.pallas_call(
      _decor_body,
      grid=grid,
      in_specs=[spec] * 5,
      out_specs=[spec] * 2,
      out_shape=[jax.ShapeDtypeStruct((8, NP8), jnp.float32)] * 2,
  )(y2, o2, ay2, ao2, f2)


def kernel(x, pos, y, theta, W1, b1, W2, b2, edge_index, edge_attr):
  f32 = jnp.float32
  i32 = jnp.int32
  sigma_sq = theta[0]
  phi = theta[1]
  tau_sq = theta[2] * theta[0]
  t02 = theta[0] + theta[2]

  o2d = _mlp(x, W1, b1.reshape(1, 128), W2.reshape(1, 128),
             jnp.reshape(b2, (1, 1)).astype(f32))
  o_flat = o2d.reshape(N)

  epad = EPAD - edge_index.shape[1]
  src = jnp.concatenate([edge_index[0], jnp.zeros((epad,), i32)])
  dst = jnp.concatenate([edge_index[1], jnp.full((epad,), N, i32)])
  slot = jnp.concatenate([edge_attr[:, 0], jnp.zeros((epad,), i32)])

  posx = pos[:, 0]
  posy = pos[:, 1]
  scal = jnp.concatenate([
      jnp.full((16,), sigma_sq, f32), jnp.full((16,), phi, f32)])

  cov_f, px_f, py_f = _sc1(src, dst, slot, posx, posy, scal)
  cov3 = cov_f.reshape(K, 8, NP8)
  px3 = px_f.reshape(K, 8, NP8)
  py3 = py_f.reshape(K, 8, NP8)

  mx, mn = _minmax(px3, py3)

  fill_key = jax.random.key(12345)
  rf = jax.random.uniform(fill_key, (N, K * 2), dtype=f32)
  rf3 = jnp.pad(rf.T.reshape(2 * K, 8, NROW),
                ((0, 0), (0, 0), (0, NP8 - NROW)), constant_values=1.0)

  th = jnp.stack([sigma_sq, phi, tau_sq, t02]).reshape(1, 4).astype(f32)
  bmat, f2 = _chol(th, mx, mn, cov3, px3, py3, rf3)

  accy, acco = _sc2(src, dst, slot, y, o_flat, bmat.reshape(KNP))

  pad2 = ((0, 0), (0, NP8 - NROW))
  y2 = jnp.pad(y.reshape(8, NROW), pad2)
  o2 = jnp.pad(o_flat.reshape(8, NROW), pad2)
  ay2 = accy.reshape(8, NP8)
  ao2 = acco.reshape(8, NP8)

  yd2, od2 = _decor(y2, o2, ay2, ao2, f2)
  y_decor = yd2[:, :NROW].reshape(N)
  o_decor = od2[:, :NROW].reshape(N)
  return (y_decor, o_decor)


# speedup vs baseline: 47.4599x; 1.1429x over previous
import functools

import jax
import jax.numpy as jnp
from jax import lax
from jax.experimental import pallas as pl
from jax.experimental.pallas import tpu as pltpu
from jax.experimental.pallas import tpu_sc as plsc

N = 50000
K = 20
NP8 = 6272
NROW = N // 8
NPAD = 8 * NP8
KNP = K * NPAD
CHUNK = 896
EPAD = 1120 * CHUNK
NCHUNK_PER_SUB = 70
EPS = 1e-12


def _sqrt16(t):
  bits = plsc.bitcast(t, jnp.int32)
  h = jnp.int32(0x5F3759DF) - lax.shift_right_logical(bits, 1)
  r = plsc.bitcast(h, jnp.float32)
  half_t = 0.5 * t
  for _ in range(3):
    r = r * (1.5 - half_t * r * r)
  return t * r


def _mlp_body(x_ref, w1_ref, b1_ref, w2t_ref, b2_ref, o_ref):
  h = jnp.dot(x_ref[...], w1_ref[...], preferred_element_type=jnp.float32)
  h = jnp.maximum(h + b1_ref[...], 0.0)
  o = lax.dot_general(w2t_ref[...], h, (((1,), (1,)), ((), ())),
                      preferred_element_type=jnp.float32)
  o_ref[...] = o + b2_ref[0, 0]


def _mlp(x, w1, b1, w2t, b2):
  bn = 512
  grid = (pl.cdiv(N, bn),)
  return pl.pallas_call(
      _mlp_body,
      grid=grid,
      in_specs=[
          pl.BlockSpec((bn, 128), lambda i: (i, 0)),
          pl.BlockSpec((128, 128), lambda i: (0, 0)),
          pl.BlockSpec((1, 128), lambda i: (0, 0)),
          pl.BlockSpec((1, 128), lambda i: (0, 0)),
          pl.BlockSpec(memory_space=pltpu.SMEM),
      ],
      out_specs=pl.BlockSpec((1, bn), lambda i: (0, i)),
      out_shape=jax.ShapeDtypeStruct((1, N), jnp.float32),
  )(x, w1, b1, w2t, b2)


KC4 = K * 4 * NP8


KC2 = K * 2 * NP8
KC1 = K * NP8


def _sc1_body(src_h, dst_h, slot_h, posx_h, posy_h, sp_h,
              cov_o, px_o, py_o,
              posx_v, posy_v, sbuf, dbuf, lbuf,
              covb, pxb, pyb, idxb, sp_v, zbuf, bnc,
              acc_cov, acc_px, acc_py):
  core = lax.axis_index("c")
  sub = lax.axis_index("s")

  pltpu.sync_copy(posx_h, posx_v)
  pltpu.sync_copy(posy_h, posy_v)
  pltpu.sync_copy(sp_h, sp_v)

  sig = sp_v[pl.ds(0, 16)]
  phi = sp_v[pl.ds(16, 16)]

  def _zb(i):
    zbuf[pl.ds(i * 16, 16)] = jnp.zeros((16,), jnp.float32)
  lax.fori_loop(0, NP8 // 16, lambda i, _: (_zb(i), 0)[1], 0)

  def _pass(p, _):
    for kk in range(10):
      @pl.when(sub == kk % 16)
      def _(kk=kk):
        pltpu.sync_copy(zbuf, acc_cov.at[pl.ds(kk * NP8, NP8)])
        pltpu.sync_copy(zbuf, acc_px.at[pl.ds(kk * NP8, NP8)])
        pltpu.sync_copy(zbuf, acc_py.at[pl.ds(kk * NP8, NP8)])
    plsc.subcore_barrier()

    row = p // 2
    half = p - 2 * row
    lvbase = 10 * half
    rbase = core * 4 + row

    def _chunk(g, _):
      off = (g * 16 + sub) * CHUNK
      pltpu.sync_copy(src_h.at[pl.ds(off, CHUNK)], sbuf)
      pltpu.sync_copy(dst_h.at[pl.ds(off, CHUNK)], dbuf)
      pltpu.sync_copy(slot_h.at[pl.ds(off, CHUNK)], lbuf)

      def _vec(j, _):
        sl = pl.ds(j * 16, 16)
        sv = sbuf[sl]
        dv = dbuf[sl]
        lv = lbuf[sl]
        dg = jnp.minimum(dv, N - 1)
        xs = plsc.load_gather(posx_v, [sv])
        ys = plsc.load_gather(posy_v, [sv])
        xd = plsc.load_gather(posx_v, [dg])
        yd = plsc.load_gather(posy_v, [dg])
        dx = xd - xs
        dy = yd - ys
        t = dx * dx + dy * dy + EPS
        dist = _sqrt16(t)
        cov = sig * jnp.exp(-phi * dist)
        r8 = dv // NROW
        c6 = dv - r8 * NROW
        lvr = lv - lvbase
        ok = (r8 == rbase) & (lvr >= 0) & (lvr < 10)
        idx = lvr * NP8 + c6
        idx = jnp.where(ok, idx, -1)
        covb[sl] = cov
        pxb[sl] = xs
        pyb[sl] = ys
        idxb[sl] = idx
        return 0

      lax.fori_loop(0, CHUNK // 16, _vec, 0)
      ind = plsc.Indices(idxb, ignored_value=-1)
      pltpu.sync_copy(covb, acc_cov.at[ind], add=True)
      pltpu.sync_copy(pxb, acc_px.at[ind], add=True)
      pltpu.sync_copy(pyb, acc_py.at[ind], add=True)
      return 0

    lax.fori_loop(0, NCHUNK_PER_SUB, _chunk, 0)
    plsc.subcore_barrier()

    for kk in range(10):
      @pl.when(sub == kk % 16)
      def _(kk=kk):
        dsto = (kk + lvbase) * 8 * NP8 + rbase * NP8
        pltpu.sync_copy(acc_cov.at[pl.ds(kk * NP8, NP8)], bnc)
        pltpu.sync_copy(bnc, cov_o.at[pl.ds(dsto, NP8)])
        pltpu.sync_copy(acc_px.at[pl.ds(kk * NP8, NP8)], bnc)
        pltpu.sync_copy(bnc, px_o.at[pl.ds(dsto, NP8)])
        pltpu.sync_copy(acc_py.at[pl.ds(kk * NP8, NP8)], bnc)
        pltpu.sync_copy(bnc, py_o.at[pl.ds(dsto, NP8)])
    plsc.subcore_barrier()
    return 0

  lax.fori_loop(0, 8, _pass, 0)


def _sc1(src, dst, slot, posx, posy, scal):
  mesh = plsc.VectorSubcoreMesh(core_axis_name="c", subcore_axis_name="s")
  f32 = jnp.float32
  kern = pl.kernel(
      _sc1_body,
      out_type=[jax.ShapeDtypeStruct((KNP,), f32)] * 3,
      mesh=mesh,
      compiler_params=pltpu.CompilerParams(needs_layout_passes=False),
      scratch_types=[
          pltpu.VMEM((N,), f32), pltpu.VMEM((N,), f32),
          pltpu.VMEM((CHUNK,), jnp.int32), pltpu.VMEM((CHUNK,), jnp.int32),
          pltpu.VMEM((CHUNK,), jnp.int32),
          pltpu.VMEM((CHUNK,), f32), pltpu.VMEM((CHUNK,), f32),
          pltpu.VMEM((CHUNK,), f32), pltpu.VMEM((CHUNK,), jnp.int32),
          pltpu.VMEM((32,), f32),
          pltpu.VMEM((NP8,), f32), pltpu.VMEM((NP8,), f32),
          pltpu.VMEM_SHARED((10 * NP8,), f32),
          pltpu.VMEM_SHARED((10 * NP8,), f32),
          pltpu.VMEM_SHARED((10 * NP8,), f32),
      ],
  )
  return kern(src, dst, slot, posx, posy, scal)


def _minmax_body(px_ref, py_ref, mx_ref, mn_ref):
  i = pl.program_id(0)
  lanes = i * 128 + lax.broadcasted_iota(jnp.int32, (K, 8, 128), 2)
  valid = lanes < NROW
  big = jnp.float32(3.4e38)
  px = px_ref[...]
  py = py_ref[...]
  bmax = jnp.maximum(jnp.max(jnp.where(valid, px, -big)),
                     jnp.max(jnp.where(valid, py, -big)))
  bmin = jnp.minimum(jnp.min(jnp.where(valid, px, big)),
                     jnp.min(jnp.where(valid, py, big)))

  @pl.when(i == 0)
  def _():
    mx_ref[0, 0] = -big
    mn_ref[0, 0] = big

  mx_ref[0, 0] = jnp.maximum(mx_ref[0, 0], bmax)
  mn_ref[0, 0] = jnp.minimum(mn_ref[0, 0], bmin)


def _minmax(px3, py3):
  grid = (NP8 // 128,)
  return pl.pallas_call(
      _minmax_body,
      grid=grid,
      in_specs=[
          pl.BlockSpec((K, 8, 128), lambda i: (0, 0, i)),
          pl.BlockSpec((K, 8, 128), lambda i: (0, 0, i)),
      ],
      out_specs=[
          pl.BlockSpec(memory_space=pltpu.SMEM),
          pl.BlockSpec(memory_space=pltpu.SMEM),
      ],
      out_shape=[jax.ShapeDtypeStruct((1, 1), jnp.float32)] * 2,
  )(px3, py3)


def _chol_body(th_ref, mx_ref, mn_ref, cov_ref, px_ref, py_ref, rf_ref,
               b_ref, f_ref):
  sig = th_ref[0, 0]
  phi = th_ref[0, 1]
  tsq = th_ref[0, 2]
  t02 = th_ref[0, 3]
  scale = 10000.0 * (mx_ref[0, 0] - mn_ref[0, 0])

  fx = []
  fy = []
  for a in range(K):
    cx = px_ref[a]
    cy = py_ref[a]
    fx.append(jnp.where(cx == 0.0, rf_ref[2 * a] * scale, cx))
    fy.append(jnp.where(cy == 0.0, rf_ref[2 * a + 1] * scale, cy))

  m = [[None] * K for _ in range(K)]
  for a in range(K):
    for b in range(a + 1):
      dx = fx[a] - fx[b]
      dy = fy[a] - fy[b]
      dist = jnp.sqrt(dx * dx + dy * dy + EPS)
      v = sig * jnp.exp(-phi * dist)
      if a == b:
        v = v + tsq
      m[a][b] = v

  inv = [None] * K
  for j in range(K):
    inv[j] = 1.0 / jnp.sqrt(m[j][j])
    for r in range(j + 1, K):
      m[r][j] = m[r][j] * inv[j]
    for r in range(j + 1, K):
      for b in range(j + 1, r + 1):
        m[r][b] = m[r][b] - m[r][j] * m[b][j]

  z = [None] * K
  for j in range(K):
    acc = cov_ref[j]
    for p in range(j):
      acc = acc - m[j][p] * z[p]
    z[j] = acc * inv[j]

  f = t02
  for j in range(K):
    f = f - z[j] * z[j]
  f_ref[...] = jnp.maximum(f, 1e-6)

  bb = [None] * K
  for j in range(K - 1, -1, -1):
    acc = z[j]
    for p in range(j + 1, K):
      acc = acc - m[p][j] * bb[p]
    bb[j] = acc * inv[j]
  for j in range(K):
    b_ref[j] = bb[j]


def _chol(th, mx, mn, cov3, px3, py3, rf3):
  grid = (NP8 // 128,)
  return pl.pallas_call(
      _chol_body,
      grid=grid,
      in_specs=[
          pl.BlockSpec(memory_space=pltpu.SMEM),
          pl.BlockSpec(memory_space=pltpu.SMEM),
          pl.BlockSpec(memory_space=pltpu.SMEM),
          pl.BlockSpec((K, 8, 128), lambda i: (0, 0, i)),
          pl.BlockSpec((K, 8, 128), lambda i: (0, 0, i)),
          pl.BlockSpec((K, 8, 128), lambda i: (0, 0, i)),
          pl.BlockSpec((2 * K, 8, 128), lambda i: (0, 0, i)),
      ],
      out_specs=[
          pl.BlockSpec((K, 8, 128), lambda i: (0, 0, i)),
          pl.BlockSpec((8, 128), lambda i: (0, i)),
      ],
      out_shape=[
          jax.ShapeDtypeStruct((K, 8, NP8), jnp.float32),
          jax.ShapeDtypeStruct((8, NP8), jnp.float32),
      ],
  )(th, mx, mn, cov3, px3, py3, rf3)


def _sc2_body(src_h, dst_h, slot_h, y_h, o_h, b_h,
              accy_o, acco_o,
              y_v, o_v, sbuf, dbuf, lbuf,
              gidx, sidx, gbuf, cyb, cob, zbuf, bnc,
              b_s, accy_s, acco_s):
  core = lax.axis_index("c")
  sub = lax.axis_index("s")

  pltpu.sync_copy(y_h, y_v)
  pltpu.sync_copy(o_h, o_v)

  def _zb(i):
    zbuf[pl.ds(i * 16, 16)] = jnp.zeros((16,), jnp.float32)
  lax.fori_loop(0, NP8 // 16, lambda i, _: (_zb(i), 0)[1], 0)

  for i in range(4):
    @pl.when(sub == i)
    def _(i=i):
      pltpu.sync_copy(zbuf, accy_s.at[pl.ds(i * NP8, NP8)])

    @pl.when(sub == 8 + i)
    def _(i=i):
      pltpu.sync_copy(zbuf, acco_s.at[pl.ds(i * NP8, NP8)])
  plsc.subcore_barrier()

  cbase = core * 4

  def _qpass(q, _):
    plsc.subcore_barrier()
    for kk in range(K):
      @pl.when(sub == kk % 16)
      def _(kk=kk):
        pltpu.sync_copy(
            b_h.at[pl.ds(kk * 8 * NP8 + (cbase + q) * NP8, NP8)], bnc)
        pltpu.sync_copy(bnc, b_s.at[pl.ds(kk * NP8, NP8)])
    plsc.subcore_barrier()

    def _chunk(g, _):
      off = (g * 16 + sub) * CHUNK
      pltpu.sync_copy(src_h.at[pl.ds(off, CHUNK)], sbuf)
      pltpu.sync_copy(dst_h.at[pl.ds(off, CHUNK)], dbuf)
      pltpu.sync_copy(slot_h.at[pl.ds(off, CHUNK)], lbuf)

      def _vec(j, _):
        sl = pl.ds(j * 16, 16)
        sv = sbuf[sl]
        dv = dbuf[sl]
        lv = lbuf[sl]
        r8 = dv // NROW
        c6 = dv - r8 * NROW
        ok = r8 == cbase + q
        gi = jnp.where(ok, lv * NP8 + c6, 0)
        si = jnp.where(ok, q * NP8 + c6, -1)
        gidx[sl] = gi
        sidx[sl] = si
        cyb[sl] = plsc.load_gather(y_v, [sv])
        cob[sl] = plsc.load_gather(o_v, [sv])
        return 0

      lax.fori_loop(0, CHUNK // 16, _vec, 0)
      pltpu.sync_copy(b_s.at[plsc.Indices(gidx)], gbuf)

      def _mul(j, _):
        sl = pl.ds(j * 16, 16)
        bv = gbuf[sl]
        cyb[sl] = bv * cyb[sl]
        cob[sl] = bv * cob[sl]
        return 0

      lax.fori_loop(0, CHUNK // 16, _mul, 0)
      ind = plsc.Indices(sidx, ignored_value=-1)
      pltpu.sync_copy(cyb, accy_s.at[ind], add=True)
      pltpu.sync_copy(cob, acco_s.at[ind], add=True)
      return 0

    lax.fori_loop(0, NCHUNK_PER_SUB, _chunk, 0)
    return 0

  lax.fori_loop(0, 4, _qpass, 0)
  plsc.subcore_barrier()

  for i in range(4):
    @pl.when(sub == i)
    def _(i=i):
      pltpu.sync_copy(accy_s.at[pl.ds(i * NP8, NP8)], zbuf)
      pltpu.sync_copy(zbuf, accy_o.at[pl.ds(core * 4 * NP8 + i * NP8, NP8)])

    @pl.when(sub == 8 + i)
    def _(i=i):
      pltpu.sync_copy(acco_s.at[pl.ds(i * NP8, NP8)], zbuf)
      pltpu.sync_copy(zbuf, acco_o.at[pl.ds(core * 4 * NP8 + i * NP8, NP8)])


def _sc2(src, dst, slot, y, o, bflat):
  mesh = plsc.VectorSubcoreMesh(core_axis_name="c", subcore_axis_name="s")
  f32 = jnp.float32
  kern = pl.kernel(
      _sc2_body,
      out_type=[jax.ShapeDtypeStruct((NPAD,), f32)] * 2,
      mesh=mesh,
      compiler_params=pltpu.CompilerParams(needs_layout_passes=False),
      scratch_types=[
          pltpu.VMEM((N,), f32), pltpu.VMEM((N,), f32),
          pltpu.VMEM((CHUNK,), jnp.int32), pltpu.VMEM((CHUNK,), jnp.int32),
          pltpu.VMEM((CHUNK,), jnp.int32),
          pltpu.VMEM((CHUNK,), jnp.int32), pltpu.VMEM((CHUNK,), jnp.int32),
          pltpu.VMEM((CHUNK,), f32), pltpu.VMEM((CHUNK,), f32),
          pltpu.VMEM((CHUNK,), f32),
          pltpu.VMEM((NP8,), f32), pltpu.VMEM((NP8,), f32),
          pltpu.VMEM_SHARED((K * NP8,), f32),
          pltpu.VMEM_SHARED((4 * NP8,), f32),
          pltpu.VMEM_SHARED((4 * NP8,), f32),
      ],
  )
  return kern(src, dst, slot, y, o, bflat)


def _decor_body(y_ref, o_ref, ay_ref, ao_ref, f_ref, yd_ref, od_ref):
  rs = jnp.sqrt(f_ref[...])
  yd_ref[...] = (y_ref[...] - ay_ref[...]) / rs
  od_ref[...] = (o_ref[...] - ao_ref[...]) / rs


def _decor(y2, o2, ay2, ao2, f2):
  grid = (NP8 // 128,)
  spec = pl.BlockSpec((8, 128), lambda i: (0, i))
  return pl.pallas_call(
      _decor_body,
      grid=grid,
      in_specs=[spec] * 5,
      out_specs=[spec] * 2,
      out_shape=[jax.ShapeDtypeStruct((8, NP8), jnp.float32)] * 2,
  )(y2, o2, ay2, ao2, f2)


def kernel(x, pos, y, theta, W1, b1, W2, b2, edge_index, edge_attr):
  f32 = jnp.float32
  i32 = jnp.int32
  sigma_sq = theta[0]
  phi = theta[1]
  tau_sq = theta[2] * theta[0]
  t02 = theta[0] + theta[2]

  o2d = _mlp(x, W1, b1.reshape(1, 128), W2.reshape(1, 128),
             jnp.reshape(b2, (1, 1)).astype(f32))
  o_flat = o2d.reshape(N)

  epad = EPAD - edge_index.shape[1]
  src = jnp.concatenate([edge_index[0], jnp.zeros((epad,), i32)])
  dst = jnp.concatenate([edge_index[1], jnp.full((epad,), N, i32)])
  slot = jnp.concatenate([edge_attr[:, 0], jnp.zeros((epad,), i32)])

  posx = pos[:, 0]
  posy = pos[:, 1]
  scal = jnp.concatenate([
      jnp.full((16,), sigma_sq, f32), jnp.full((16,), phi, f32)])

  cov_f, px_f, py_f = _sc1(src, dst, slot, posx, posy, scal)
  cov3 = cov_f.reshape(K, 8, NP8)
  px3 = px_f.reshape(K, 8, NP8)
  py3 = py_f.reshape(K, 8, NP8)

  mx, mn = _minmax(px3, py3)

  fill_key = jax.random.key(12345)
  rf = jax.random.uniform(fill_key, (N, K * 2), dtype=f32)
  rf3 = jnp.pad(rf.T.reshape(2 * K, 8, NROW),
                ((0, 0), (0, 0), (0, NP8 - NROW)), constant_values=1.0)

  th = jnp.stack([sigma_sq, phi, tau_sq, t02]).reshape(1, 4).astype(f32)
  bmat, f2 = _chol(th, mx, mn, cov3, px3, py3, rf3)

  accy, acco = _sc2(src, dst, slot, y, o_flat, bmat.reshape(KNP))

  pad2 = ((0, 0), (0, NP8 - NROW))
  y2 = jnp.pad(y.reshape(8, NROW), pad2)
  o2 = jnp.pad(o_flat.reshape(8, NROW), pad2)
  ay2 = accy.reshape(8, NP8)
  ao2 = acco.reshape(8, NP8)

  yd2, od2 = _decor(y2, o2, ay2, ao2, f2)
  y_decor = yd2[:, :NROW].reshape(N)
  o_decor = od2[:, :NROW].reshape(N)
  return (y_decor, o_decor)
